# baseline (speedup 1.0000x reference)
import functools

import jax
import jax.numpy as jnp
from jax import lax
from jax.experimental import pallas as pl
from jax.experimental.pallas import tpu as pltpu
from jax.experimental.pallas import tpu_sc as plsc

NC = 2
NS = 16
CHUNK = 128


def _sc_mesh():
    return plsc.VectorSubcoreMesh(core_axis_name="c", subcore_axis_name="s")


def _make_deg_kernel(npad, cpt, rpt):
    @functools.partial(
        pl.kernel,
        out_type=jax.ShapeDtypeStruct((npad,), jnp.float32),
        mesh=_sc_mesh(),
        scratch_types=[
            pltpu.VMEM((cpt, 2, CHUNK), jnp.int32),
            pltpu.VMEM((CHUNK,), jnp.float32),
            pltpu.VMEM((rpt,), jnp.float32),
            pltpu.VMEM_SHARED((npad,), jnp.float32),
        ],
    )
    def deg_kernel(eidx_hbm, h_hbm, idx_v, ones_v, stage_v, acc):
        c = lax.axis_index("c")
        s = lax.axis_index("s")

        @pl.when(c == 0)
        def _():
            pltpu.sync_copy(eidx_hbm.at[s], idx_v)
            for m in range(CHUNK // 16):
                ones_v[pl.ds(16 * m, 16)] = jnp.ones((16,), jnp.float32)

            @pl.loop(0, rpt, step=16)
            def _(r):
                stage_v[pl.ds(r, 16)] = jnp.zeros((16,), jnp.float32)

            pltpu.sync_copy(stage_v, acc.at[pl.ds(s * rpt, rpt)])
            plsc.subcore_barrier()

            @pl.loop(0, cpt)
            def _(j):
                pltpu.sync_copy(ones_v, acc.at[idx_v.at[j].at[1]], add=True)

            plsc.subcore_barrier()
            pltpu.sync_copy(acc.at[pl.ds(s * rpt, rpt)], stage_v)
            pltpu.sync_copy(stage_v, h_hbm.at[pl.ds(s * rpt, rpt)])

    return deg_kernel


def _make_agg_kernel(npad, cpt, rpt, d):
    assert rpt % CHUNK == 0 and cpt >= 8 and cpt % 4 == 0

    @functools.partial(
        pl.kernel,
        out_type=jax.ShapeDtypeStruct((npad, d), jnp.float32),
        mesh=_sc_mesh(),
        scratch_types=[
            pltpu.VMEM((4, 2, CHUNK), jnp.int32),
            pltpu.VMEM((2, CHUNK, d), jnp.float32),
            pltpu.VMEM_SHARED((npad, d), jnp.float32),
            pltpu.SemaphoreType.DMA,
            pltpu.SemaphoreType.DMA,
            pltpu.SemaphoreType.DMA,
            pltpu.SemaphoreType.DMA,
        ],
    )
    def agg_kernel(eidx_hbm, xs_hbm, p_hbm,
                   ring, buf, acc, sem0, sem1, semi0, semi1):
        c = lax.axis_index("c")
        s = lax.axis_index("s")
        sems = (sem0, sem1)
        semis = (semi0, semi1)

        @pl.when(c == 0)
        def _():
            for r in range(CHUNK):
                for m in range(d // 16):
                    buf[0, r, pl.ds(16 * m, 16)] = jnp.zeros((16,), jnp.float32)
            for r in range(0, rpt, CHUNK):
                pltpu.async_copy(buf.at[0], acc.at[pl.ds(s * rpt + r, CHUNK)], sem0)
            for r in range(0, rpt, CHUNK):
                pltpu.make_async_copy(buf.at[0], acc.at[pl.ds(0, CHUNK)], sem0).wait()

            plsc.subcore_barrier()

            def prefetch_idx(jj, k, b):
                pltpu.async_copy(eidx_hbm.at[s].at[jj], ring.at[k], semis[b])

            def wait_idx(b):
                pltpu.make_async_copy(eidx_hbm.at[s].at[0], ring.at[0], semis[b]).wait()

            def issue_gather(k, b):
                pltpu.async_copy(xs_hbm.at[ring.at[k].at[0]], buf.at[b], sems[b])

            def wait_gather(b):
                pltpu.make_async_copy(xs_hbm.at[ring.at[0].at[0]], buf.at[b], sems[b]).wait()

            def scatter(k, b):
                pltpu.sync_copy(buf.at[b], acc.at[ring.at[k].at[1]], add=True)

            pltpu.sync_copy(eidx_hbm.at[s].at[0], ring.at[0])
            pltpu.sync_copy(eidx_hbm.at[s].at[1], ring.at[1])
            prefetch_idx(2, 2, 0)
            prefetch_idx(3, 3, 1)
            issue_gather(0, 0)
            issue_gather(1, 1)

            @pl.loop(0, cpt - 4, step=4)
            def _(g):
                for k in range(4):
                    b = k % 2
                    wait_gather(b)
                    scatter(k, b)
                    wait_idx(b)
                    issue_gather((k + 2) % 4, b)
                    prefetch_idx(g + k + 4, k, b)

            for k in range(4):
                b = k % 2
                j = cpt - 4 + k
                wait_gather(b)
                scatter(k, b)
                if j + 2 < cpt:
                    wait_idx(b)
                    issue_gather((k + 2) % 4, b)

            plsc.subcore_barrier()

            nwb = rpt // CHUNK
            pltpu.async_copy(acc.at[pl.ds(s * rpt, CHUNK)], buf.at[0], sem0)
            for i in range(nwb):
                b = i % 2
                pltpu.make_async_copy(acc.at[pl.ds(0, CHUNK)], buf.at[b], sems[b]).wait()
                if i + 1 < nwb:
                    pltpu.async_copy(acc.at[pl.ds(s * rpt + (i + 1) * CHUNK, CHUNK)],
                                     buf.at[1 - b], sems[1 - b])
                pltpu.sync_copy(buf.at[b],
                                p_hbm.at[pl.ds(s * rpt + i * CHUNK, CHUNK)])

    return agg_kernel


def _pack_edges(src, dst, n, npad, cpt):
    e = src.shape[0]
    cap = NS * cpt * CHUNK
    padlen = cap - e
    s_part = jnp.concatenate([src, jnp.zeros((padlen,), jnp.int32)])
    d_part = jnp.concatenate(
        [dst, n + (jnp.arange(padlen, dtype=jnp.int32) % (npad - n))])
    return jnp.stack(
        [s_part.reshape(NS, cpt, CHUNK), d_part.reshape(NS, cpt, CHUNK)], axis=2)


def _scale_body(x_ref, dv_ref, o_ref):
    o_ref[...] = x_ref[...] * dv_ref[...]


def _layer1_body(p_ref, xs_ref, dv_ref, w_ref, b_ref, o_ref):
    g = (p_ref[...] + xs_ref[...]) * dv_ref[...]
    h = jnp.dot(g, w_ref[...], preferred_element_type=jnp.float32) + b_ref[...]
    o_ref[...] = jnp.maximum(h, 0.0) * dv_ref[...]


def _layer23_body(q_ref, hs_ref, dv_ref, w1_ref, b1_ref,
                  w2_ref, b2_ref, o1_ref, o2_ref):
    g = (q_ref[...] + hs_ref[...]) * dv_ref[...]
    o1_ref[...] = jnp.dot(g, w1_ref[...], preferred_element_type=jnp.float32) + b1_ref[...]
    o2_ref[...] = jnp.dot(g, w2_ref[...], preferred_element_type=jnp.float32) + b2_ref[...]


def kernel(x, edge_index, W0, b0, W1, b1, W2, b2):
    n, d = x.shape
    e = edge_index.shape[1]

    npad = ((n + 1 + 128 * NS - 1) // (128 * NS)) * (128 * NS)
    cpt = -(-e // (NS * CHUNK))
    cpt = ((cpt + 3) // 4) * 4

    src = edge_index[0].astype(jnp.int32)
    dst = edge_index[1].astype(jnp.int32)
    eidx = _pack_edges(src, dst, n, npad, cpt)

    hist = _make_deg_kernel(npad, cpt, npad // NS)(eidx)
    deg = hist[:n] + 1.0
    dv = lax.rsqrt(deg)[:, None]

    agg = _make_agg_kernel(npad, cpt, npad // NS, d)

    grid_r = 1000
    grid = (n // grid_r,)
    row_spec = pl.BlockSpec((grid_r, d), lambda i: (i, 0))
    col_spec = pl.BlockSpec((grid_r, 1), lambda i: (i, 0))
    w_spec = pl.BlockSpec((d, d), lambda i: (0, 0))
    b_spec = pl.BlockSpec((1, d), lambda i: (0, 0))
    out_nd = jax.ShapeDtypeStruct((n, d), jnp.float32)

    xs = pl.pallas_call(
        _scale_body,
        grid=grid,
        in_specs=[row_spec, col_spec],
        out_specs=row_spec,
        out_shape=out_nd,
    )(x, dv)

    p = agg(eidx, xs)

    hs0 = pl.pallas_call(
        _layer1_body,
        grid=grid,
        in_specs=[row_spec, row_spec, col_spec, w_spec, b_spec],
        out_specs=row_spec,
        out_shape=out_nd,
    )(p[:n], xs, dv, W0, b0.reshape(1, d))

    q = agg(eidx, hs0)

    x_, x2 = pl.pallas_call(
        _layer23_body,
        grid=grid,
        in_specs=[row_spec, row_spec, col_spec,
                  w_spec, b_spec, w_spec, b_spec],
        out_specs=(row_spec, row_spec),
        out_shape=(out_nd, out_nd),
    )(q[:n], hs0, dv, W1, b1.reshape(1, d), W2, b2.reshape(1, d))

    return (x_, x2)


# speedup vs baseline: 1.0153x; 1.0153x over previous
import functools

import jax
import jax.numpy as jnp
from jax import lax
from jax.experimental import pallas as pl
from jax.experimental.pallas import tpu as pltpu
from jax.experimental.pallas import tpu_sc as plsc

NC = 2
NS = 16
CHUNK = 128


def _sc_mesh():
    return plsc.VectorSubcoreMesh(core_axis_name="c", subcore_axis_name="s")


def _make_deg_kernel(npad, cpt, rpt):
    @functools.partial(
        pl.kernel,
        out_type=jax.ShapeDtypeStruct((npad,), jnp.float32),
        mesh=_sc_mesh(),
        scratch_types=[
            pltpu.VMEM((cpt, CHUNK), jnp.int32),
            pltpu.VMEM((CHUNK,), jnp.float32),
            pltpu.VMEM((rpt,), jnp.float32),
            pltpu.VMEM_SHARED((npad,), jnp.float32),
        ],
    )
    def deg_kernel(dst_hbm, h_hbm, idx_v, ones_v, stage_v, acc):
        c = lax.axis_index("c")
        s = lax.axis_index("s")

        @pl.when(c == 0)
        def _():
            pltpu.sync_copy(dst_hbm.at[s], idx_v)
            for m in range(CHUNK // 16):
                ones_v[pl.ds(16 * m, 16)] = jnp.ones((16,), jnp.float32)

            @pl.loop(0, rpt, step=16)
            def _(r):
                stage_v[pl.ds(r, 16)] = jnp.zeros((16,), jnp.float32)

            pltpu.sync_copy(stage_v, acc.at[pl.ds(s * rpt, rpt)])
            plsc.subcore_barrier()

            @pl.loop(0, cpt)
            def _(j):
                pltpu.sync_copy(ones_v, acc.at[idx_v.at[j]], add=True)

            plsc.subcore_barrier()
            pltpu.sync_copy(acc.at[pl.ds(s * rpt, rpt)], stage_v)
            pltpu.sync_copy(stage_v, h_hbm.at[pl.ds(s * rpt, rpt)])

    return deg_kernel


def _make_agg_kernel(npad, cpt, rpt, d):
    assert rpt % CHUNK == 0 and cpt >= 8 and cpt % 4 == 0

    @functools.partial(
        pl.kernel,
        out_type=jax.ShapeDtypeStruct((npad, d), jnp.float32),
        mesh=_sc_mesh(),
        scratch_types=[
            pltpu.VMEM((4, CHUNK), jnp.int32),
            pltpu.VMEM((4, CHUNK), jnp.int32),
            pltpu.VMEM((2, CHUNK, d), jnp.float32),
            pltpu.VMEM_SHARED((npad, d), jnp.float32),
            pltpu.SemaphoreType.DMA,
            pltpu.SemaphoreType.DMA,
            pltpu.SemaphoreType.DMA,
            pltpu.SemaphoreType.DMA,
        ],
    )
    def agg_kernel(src_hbm, dst_hbm, xs_hbm, p_hbm,
                   sring, dring, buf, acc, sem0, sem1, semi0, semi1):
        c = lax.axis_index("c")
        s = lax.axis_index("s")
        sems = (sem0, sem1)
        semis = (semi0, semi1)

        @pl.when(c == 0)
        def _():
            for r in range(CHUNK):
                for m in range(d // 16):
                    buf[0, r, pl.ds(16 * m, 16)] = jnp.zeros((16,), jnp.float32)
            for r in range(0, rpt, CHUNK):
                pltpu.async_copy(buf.at[0], acc.at[pl.ds(s * rpt + r, CHUNK)], sem0)
            for r in range(0, rpt, CHUNK):
                pltpu.make_async_copy(buf.at[0], acc.at[pl.ds(0, CHUNK)], sem0).wait()

            plsc.subcore_barrier()

            def prefetch_idx(jj, k, b):
                pltpu.async_copy(src_hbm.at[s].at[jj], sring.at[k], semis[b])
                pltpu.async_copy(dst_hbm.at[s].at[jj], dring.at[k], semis[b])

            def wait_idx(b):
                pltpu.make_async_copy(src_hbm.at[s].at[0], sring.at[0], semis[b]).wait()
                pltpu.make_async_copy(dst_hbm.at[s].at[0], dring.at[0], semis[b]).wait()

            def issue_gather(k, b):
                pltpu.async_copy(xs_hbm.at[sring.at[k]], buf.at[b], sems[b])

            def wait_gather(b):
                pltpu.make_async_copy(xs_hbm.at[sring.at[0]], buf.at[b], sems[b]).wait()

            def scatter(k, b):
                pltpu.sync_copy(buf.at[b], acc.at[dring.at[k]], add=True)

            pltpu.sync_copy(src_hbm.at[s].at[0], sring.at[0])
            pltpu.sync_copy(dst_hbm.at[s].at[0], dring.at[0])
            pltpu.sync_copy(src_hbm.at[s].at[1], sring.at[1])
            pltpu.sync_copy(dst_hbm.at[s].at[1], dring.at[1])
            prefetch_idx(2, 2, 0)
            prefetch_idx(3, 3, 1)
            issue_gather(0, 0)
            issue_gather(1, 1)

            @pl.loop(0, cpt - 4, step=4)
            def _(g):
                for k in range(4):
                    b = k % 2
                    wait_gather(b)
                    scatter(k, b)
                    wait_idx(b)
                    issue_gather((k + 2) % 4, b)
                    prefetch_idx(g + k + 4, k, b)

            for k in range(4):
                b = k % 2
                j = cpt - 4 + k
                wait_gather(b)
                scatter(k, b)
                if j + 2 < cpt:
                    wait_idx(b)
                    issue_gather((k + 2) % 4, b)

            plsc.subcore_barrier()

            nwb = rpt // CHUNK
            pltpu.async_copy(acc.at[pl.ds(s * rpt, CHUNK)], buf.at[0], sem0)
            for i in range(nwb):
                b = i % 2
                pltpu.make_async_copy(acc.at[pl.ds(0, CHUNK)], buf.at[b], sems[b]).wait()
                if i + 1 < nwb:
                    pltpu.async_copy(acc.at[pl.ds(s * rpt + (i + 1) * CHUNK, CHUNK)],
                                     buf.at[1 - b], sems[1 - b])
                pltpu.sync_copy(buf.at[b],
                                p_hbm.at[pl.ds(s * rpt + i * CHUNK, CHUNK)])

    return agg_kernel


def _pack_edges(src, dst, n, npad, cpt):
    e = src.shape[0]
    cap = NS * cpt * CHUNK
    padlen = cap - e
    s_part = jnp.concatenate([src, jnp.zeros((padlen,), jnp.int32)])
    d_part = jnp.concatenate(
        [dst, n + (jnp.arange(padlen, dtype=jnp.int32) % (npad - n))])
    return s_part.reshape(NS, cpt, CHUNK), d_part.reshape(NS, cpt, CHUNK)


def _scale_body(x_ref, dv_ref, o_ref):
    o_ref[...] = x_ref[...] * dv_ref[...]


def _layer1_body(p_ref, xs_ref, dv_ref, w_ref, b_ref, o_ref):
    g = (p_ref[...] + xs_ref[...]) * dv_ref[...]
    h = jnp.dot(g, w_ref[...], preferred_element_type=jnp.float32) + b_ref[...]
    o_ref[...] = jnp.maximum(h, 0.0) * dv_ref[...]


def _layer23_body(q_ref, hs_ref, dv_ref, w1_ref, b1_ref,
                  w2_ref, b2_ref, o1_ref, o2_ref):
    g = (q_ref[...] + hs_ref[...]) * dv_ref[...]
    o1_ref[...] = jnp.dot(g, w1_ref[...], preferred_element_type=jnp.float32) + b1_ref[...]
    o2_ref[...] = jnp.dot(g, w2_ref[...], preferred_element_type=jnp.float32) + b2_ref[...]


def kernel(x, edge_index, W0, b0, W1, b1, W2, b2):
    n, d = x.shape
    e = edge_index.shape[1]

    npad = ((n + 1 + 128 * NS - 1) // (128 * NS)) * (128 * NS)
    cpt = -(-e // (NS * CHUNK))
    cpt = ((cpt + 3) // 4) * 4

    src = edge_index[0].astype(jnp.int32)
    dst = edge_index[1].astype(jnp.int32)
    src_p, dst_p = _pack_edges(src, dst, n, npad, cpt)

    hist = _make_deg_kernel(npad, cpt, npad // NS)(dst_p)
    deg = hist[:n] + 1.0
    dv = lax.rsqrt(deg)[:, None]

    agg = _make_agg_kernel(npad, cpt, npad // NS, d)

    grid_r = 1000
    grid = (n // grid_r,)
    row_spec = pl.BlockSpec((grid_r, d), lambda i: (i, 0))
    col_spec = pl.BlockSpec((grid_r, 1), lambda i: (i, 0))
    w_spec = pl.BlockSpec((d, d), lambda i: (0, 0))
    b_spec = pl.BlockSpec((1, d), lambda i: (0, 0))
    out_nd = jax.ShapeDtypeStruct((n, d), jnp.float32)

    xs = pl.pallas_call(
        _scale_body,
        grid=grid,
        in_specs=[row_spec, col_spec],
        out_specs=row_spec,
        out_shape=out_nd,
    )(x, dv)

    p = agg(src_p, dst_p, xs)

    hs0 = pl.pallas_call(
        _layer1_body,
        grid=grid,
        in_specs=[row_spec, row_spec, col_spec, w_spec, b_spec],
        out_specs=row_spec,
        out_shape=out_nd,
    )(p[:n], xs, dv, W0, b0.reshape(1, d))

    q = agg(src_p, dst_p, hs0)

    x_, x2 = pl.pallas_call(
        _layer23_body,
        grid=grid,
        in_specs=[row_spec, row_spec, col_spec,
                  w_spec, b_spec, w_spec, b_spec],
        out_specs=(row_spec, row_spec),
        out_shape=(out_nd, out_nd),
    )(q[:n], hs0, dv, W1, b1.reshape(1, d), W2, b2.reshape(1, d))

    return (x_, x2)
